# baseline (speedup 1.0000x reference)
import functools

import jax
import jax.numpy as jnp
from jax import lax
from jax.experimental import pallas as pl
from jax.experimental.pallas import tpu as pltpu
from jax.experimental.pallas import tpu_sc as plsc

L = 16


def _transpose_block(blk_v, line_v, iota, ncols, line_off):
    for c in range(ncols):
        col = plsc.load_gather(blk_v, [iota, jnp.full((L,), c, jnp.int32)])
        li, m = line_off + c // 8, c % 8
        line_v[li, pl.ds(m * L, L)] = col


@functools.lru_cache(maxsize=None)
def _build_transpose(V, K):
    info = plsc.get_sparse_core_info()
    NC, NS = info.num_cores, info.num_subcores
    NW = NC * NS
    Vpad = -(-V // 128) * 128
    n_lines = Vpad // 8
    n_tc = Vpad // 128
    full_tc = n_tc - 1
    tail_cols = V - full_tc * 128
    T = 4
    n_groups = full_tc // T
    n_iters = -(-n_groups // NW)
    mesh = plsc.VectorSubcoreMesh(core_axis_name="c", subcore_axis_name="s")

    @functools.partial(
        pl.kernel,
        mesh=mesh,
        out_type=jax.ShapeDtypeStruct((n_lines, 128), jnp.float32),
        compiler_params=pltpu.CompilerParams(
            use_tc_tiling_on_sc=True, needs_layout_passes=False),
        scratch_types=[
            pltpu.VMEM((K, T * 128), jnp.float32),
            pltpu.VMEM((T * 16, 128), jnp.float32),
        ],
    )
    def transpose_kernel(tab_hbm, tail_hbm, r_hbm, blk_v, line_v):
        wid = lax.axis_index("s") * NC + lax.axis_index("c")
        iota = lax.iota(jnp.int32, L)

        def body(i, carry):
            g = i * NW + wid

            @pl.when(g < n_groups)
            def _():
                c0 = g * (T * 128)
                pltpu.sync_copy(tab_hbm.at[:, pl.ds(c0, T * 128)], blk_v)
                _transpose_block(blk_v, line_v, iota, T * 128, 0)
                pltpu.sync_copy(line_v, r_hbm.at[pl.ds(g * T * 16, T * 16), :])

            return carry

        lax.fori_loop(0, n_iters, body, 0)

        @pl.when(wid == 1)
        def _():
            pltpu.sync_copy(tail_hbm,
                            r_hbm.at[pl.ds(full_tc * 16, 16), :])

    return transpose_kernel


@functools.lru_cache(maxsize=None)
def _build_gather(N, V, K):
    info = plsc.get_sparse_core_info()
    NC, NS = info.num_cores, info.num_subcores
    NW = NC * NS
    assert N % NW == 0
    n_rows = N // NW
    C = 3328
    assert n_rows % C == 0
    n_chunks = n_rows // C
    G = 128
    n_sub = C // G
    assert K == L

    mesh = plsc.VectorSubcoreMesh(core_axis_name="c", subcore_axis_name="s")

    @functools.partial(
        pl.kernel,
        mesh=mesh,
        out_type=jax.ShapeDtypeStruct((N, K), jnp.float32),
        compiler_params=pltpu.CompilerParams(use_tc_tiling_on_sc=False),
        scratch_types=[
            pltpu.VMEM((n_rows,), jnp.int32),
            pltpu.VMEM((n_rows,), jnp.float32),
            pltpu.VMEM((C, K), jnp.float32),
            pltpu.SemaphoreType.DMA,
        ],
    )
    def sc_kernel(idx_hbm, val_hbm, table_hbm, out_hbm, idx_v, val_v, rows_v, sem):
        wid = lax.axis_index("s") * NC + lax.axis_index("c")
        base = wid * n_rows
        pltpu.sync_copy(idx_hbm.at[pl.ds(base, n_rows)], idx_v)
        pltpu.sync_copy(val_hbm.at[pl.ds(base, n_rows)], val_v)
        for c in range(n_chunks):
            coff = c * C
            cps = [
                pltpu.async_copy(
                    table_hbm.at[idx_v.at[pl.ds(coff + g * G, G)]],
                    rows_v.at[pl.ds(g * G, G)],
                    sem,
                )
                for g in range(n_sub)
            ]
            for cp in cps:
                cp.wait()

            def mul_body(i, carry, coff=coff):
                r0 = i * L
                val16 = val_v[pl.ds(coff + r0, L)]
                for j in range(L):
                    r = r0 + j
                    vj = jnp.full((L,), val16[j])
                    rows_v[r, :] = rows_v[r, :] * vj
                return carry

            lax.fori_loop(0, C // L, mul_body, 0)
            pltpu.sync_copy(rows_v, out_hbm.at[pl.ds(base + coff, C)])

    return sc_kernel


def kernel(nonzero_index, nonzero_value, table):
    B, F = nonzero_index.shape
    V, K = table.shape
    N = B * F
    Vpad = -(-V // 128) * 128
    tail0 = (Vpad - 128)
    tail = jnp.pad(table[tail0:], ((0, 128 - (V - tail0)), (0, 0)))
    tail_lines = tail.reshape(16, 8 * K)
    rows = _build_transpose(V, K)(table.T, tail_lines)
    table_rows = rows.reshape(Vpad, K)
    idx = nonzero_index.reshape(N).astype(jnp.int32)
    val = nonzero_value.reshape(N)
    out = _build_gather(N, Vpad, K)(idx, val, table_rows)
    return out.reshape(B, F, K)


# speedup vs baseline: 1.1071x; 1.1071x over previous
import functools

import jax
import jax.numpy as jnp
from jax import lax
from jax.experimental import pallas as pl
from jax.experimental.pallas import tpu as pltpu
from jax.experimental.pallas import tpu_sc as plsc

L = 16


def _transpose_block(blk_v, line_v, iota, ncols, line_off):
    for c in range(ncols):
        col = plsc.load_gather(blk_v, [iota, jnp.full((L,), c, jnp.int32)])
        li, m = line_off + c // 8, c % 8
        line_v[li, pl.ds(m * L, L)] = col


@functools.lru_cache(maxsize=None)
def _build_transpose(V, K):
    info = plsc.get_sparse_core_info()
    NC, NS = info.num_cores, info.num_subcores
    NW = NC * NS
    Vpad = -(-V // 128) * 128
    n_lines = Vpad // 8
    n_tc = Vpad // 128
    full_tc = n_tc - 1
    tail_cols = V - full_tc * 128
    T = 2
    n_groups = full_tc // T
    n_iters = -(-n_groups // NW)
    mesh = plsc.VectorSubcoreMesh(core_axis_name="c", subcore_axis_name="s")

    n_pairs = -(-n_iters // 2)

    @functools.partial(
        pl.kernel,
        mesh=mesh,
        out_type=jax.ShapeDtypeStruct((n_lines, 128), jnp.float32),
        compiler_params=pltpu.CompilerParams(
            use_tc_tiling_on_sc=True, needs_layout_passes=False),
        scratch_types=[
            pltpu.VMEM((K, T * 128), jnp.float32),
            pltpu.VMEM((K, T * 128), jnp.float32),
            pltpu.VMEM((T * 16, 128), jnp.float32),
            pltpu.VMEM((T * 16, 128), jnp.float32),
            pltpu.SemaphoreType.DMA,
            pltpu.SemaphoreType.DMA,
            pltpu.SemaphoreType.DMA,
            pltpu.SemaphoreType.DMA,
        ],
    )
    def transpose_kernel(tab_hbm, tail_hbm, r_hbm,
                         blk_a, blk_b, line_a, line_b,
                         sin_a, sin_b, sout_a, sout_b):
        wid = lax.axis_index("s") * NC + lax.axis_index("c")
        iota = lax.iota(jnp.int32, L)
        blks = (blk_a, blk_b)
        lines = (line_a, line_b)
        sins = (sin_a, sin_b)
        souts = (sout_a, sout_b)

        def fire_in(j, buf, sem):
            g = j * NW + wid

            @pl.when(g < n_groups)
            def _():
                pltpu.async_copy(
                    tab_hbm.at[:, pl.ds(g * (T * 128), T * 128)], buf, sem)

        fire_in(0, blk_a, sin_a)

        def body(p, carry):
            for par in range(2):
                j = p * 2 + par
                g = j * NW + wid
                fire_in(j + 1, blks[1 - par], sins[1 - par])

                @pl.when(g < n_groups)
                def _(par=par, g=g):
                    pltpu.make_async_copy(
                        tab_hbm.at[:, pl.ds(g * (T * 128), T * 128)],
                        blks[par], sins[par]).wait()

                    @pl.when(p > 0)
                    def _():
                        pltpu.make_async_copy(
                            lines[par],
                            r_hbm.at[pl.ds(0, T * 16), :],
                            souts[par]).wait()

                    _transpose_block(blks[par], lines[par], iota, T * 128, 0)
                    pltpu.async_copy(
                        lines[par],
                        r_hbm.at[pl.ds(g * T * 16, T * 16), :], souts[par])

            return carry

        lax.fori_loop(0, n_pairs, body, 0)
        for par in range(2):
            j_last = (n_pairs - 1) * 2 + par
            g_last = j_last * NW + wid

            @pl.when(g_last < n_groups)
            def _(par=par, g_last=g_last):
                pltpu.make_async_copy(
                    lines[par],
                    r_hbm.at[pl.ds(g_last * T * 16, T * 16), :],
                    souts[par]).wait()

        @pl.when(wid == 1)
        def _():
            pltpu.sync_copy(tail_hbm,
                            r_hbm.at[pl.ds(full_tc * 16, 16), :])

    return transpose_kernel


@functools.lru_cache(maxsize=None)
def _build_gather(N, V, K):
    info = plsc.get_sparse_core_info()
    NC, NS = info.num_cores, info.num_subcores
    NW = NC * NS
    assert N % NW == 0
    n_rows = N // NW
    C = 3328
    assert n_rows % C == 0
    n_chunks = n_rows // C
    G = 128
    n_sub = C // G
    assert K == L

    mesh = plsc.VectorSubcoreMesh(core_axis_name="c", subcore_axis_name="s")

    @functools.partial(
        pl.kernel,
        mesh=mesh,
        out_type=jax.ShapeDtypeStruct((N, K), jnp.float32),
        compiler_params=pltpu.CompilerParams(use_tc_tiling_on_sc=False),
        scratch_types=[
            pltpu.VMEM((n_rows,), jnp.int32),
            pltpu.VMEM((n_rows,), jnp.float32),
            pltpu.VMEM((C, K), jnp.float32),
            pltpu.SemaphoreType.DMA,
        ],
    )
    def sc_kernel(idx_hbm, val_hbm, table_hbm, out_hbm, idx_v, val_v, rows_v, sem):
        wid = lax.axis_index("s") * NC + lax.axis_index("c")
        base = wid * n_rows
        pltpu.sync_copy(idx_hbm.at[pl.ds(base, n_rows)], idx_v)
        pltpu.sync_copy(val_hbm.at[pl.ds(base, n_rows)], val_v)
        for c in range(n_chunks):
            coff = c * C
            cps = [
                pltpu.async_copy(
                    table_hbm.at[idx_v.at[pl.ds(coff + g * G, G)]],
                    rows_v.at[pl.ds(g * G, G)],
                    sem,
                )
                for g in range(n_sub)
            ]
            for cp in cps:
                cp.wait()

            def mul_body(i, carry, coff=coff):
                r0 = i * L
                val16 = val_v[pl.ds(coff + r0, L)]
                for j in range(L):
                    r = r0 + j
                    vj = jnp.full((L,), val16[j])
                    rows_v[r, :] = rows_v[r, :] * vj
                return carry

            lax.fori_loop(0, C // L, mul_body, 0)
            pltpu.sync_copy(rows_v, out_hbm.at[pl.ds(base + coff, C)])

    return sc_kernel


def kernel(nonzero_index, nonzero_value, table):
    B, F = nonzero_index.shape
    V, K = table.shape
    N = B * F
    Vpad = -(-V // 128) * 128
    tail0 = (Vpad - 128)
    tail = jnp.pad(table[tail0:], ((0, 128 - (V - tail0)), (0, 0)))
    tail_lines = tail.reshape(16, 8 * K)
    rows = _build_transpose(V, K)(table.T, tail_lines)
    table_rows = rows.reshape(Vpad, K)
    idx = nonzero_index.reshape(N).astype(jnp.int32)
    val = nonzero_value.reshape(N)
    out = _build_gather(N, Vpad, K)(idx, val, table_rows)
    return out.reshape(B, F, K)


# speedup vs baseline: 1.7514x; 1.5820x over previous
import functools

import jax
import jax.numpy as jnp
from jax import lax
from jax.experimental import pallas as pl
from jax.experimental.pallas import tpu as pltpu
from jax.experimental.pallas import tpu_sc as plsc

L = 16


@functools.lru_cache(maxsize=None)
def _build_gather(B, F, V, K):
    info = plsc.get_sparse_core_info()
    NC, NS = info.num_cores, info.num_subcores
    NW = NC * NS
    assert B % (NW * 128) == 0 and K == L
    CB = B // NW
    TC = CB // 128
    G = 128
    n_sub = CB // G

    mesh = plsc.VectorSubcoreMesh(core_axis_name="c", subcore_axis_name="s")

    @functools.partial(
        pl.kernel,
        mesh=mesh,
        out_type=jax.ShapeDtypeStruct((F, K // 8, B // 128, 8, 128),
                                      jnp.float32),
        compiler_params=pltpu.CompilerParams(
            use_tc_tiling_on_sc=False, needs_layout_passes=False),
        scratch_types=[
            pltpu.VMEM((CB,), jnp.int32),
            pltpu.VMEM((CB,), jnp.float32),
            pltpu.VMEM((CB, K), jnp.float32),
            pltpu.VMEM((K // 8, TC, 8, 128), jnp.float32),
            pltpu.SemaphoreType.DMA,
        ],
    )
    def sc_kernel(idx_hbm, val_hbm, table_hbm, out_hbm,
                  idx_v, val_v, rows_v, outp_v, sem):
        wid = lax.axis_index("s") * NC + lax.axis_index("c")
        b0 = wid * CB
        iota = lax.iota(jnp.int32, L)

        def fbody(f, carry):
            pltpu.sync_copy(idx_hbm.at[f, pl.ds(b0, CB)], idx_v)
            pltpu.sync_copy(val_hbm.at[f, pl.ds(b0, CB)], val_v)
            cps = [
                pltpu.async_copy(
                    table_hbm.at[idx_v.at[pl.ds(g * G, G)]],
                    rows_v.at[pl.ds(g * G, G)],
                    sem,
                )
                for g in range(n_sub)
            ]
            for cp in cps:
                cp.wait()
            for j in range(CB // L):
                r16 = j * L + iota
                val16 = val_v[pl.ds(j * L, L)]
                for k in range(K):
                    col = plsc.load_gather(
                        rows_v, [r16, jnp.full((L,), k, jnp.int32)])
                    outp_v[k // 8, j // 8, k % 8,
                           pl.ds((j % 8) * L, L)] = col * val16
            pltpu.sync_copy(
                outp_v,
                out_hbm.at[f, :, pl.ds(wid * TC, TC), :, :])
            return carry

        lax.fori_loop(0, F, fbody, 0)

    return sc_kernel


def kernel(nonzero_index, nonzero_value, table):
    B, F = nonzero_index.shape
    V, K = table.shape
    idxt = nonzero_index.T.astype(jnp.int32)
    valt = nonzero_value.T
    res = _build_gather(B, F, V, K)(idxt, valt, table)
    out = res.transpose((2, 4, 0, 1, 3)).reshape(B, F, K)
    return out
